# Initial kernel scaffold; baseline (speedup 1.0000x reference)
import functools

import jax
import jax.numpy as jnp
from jax import lax
from jax.experimental import pallas as pl
from jax.experimental.pallas import tpu as pltpu
from jax.experimental.pallas import tpu_sc as plsc

N_NODES = 10000
D_FEAT = 128
N_EDGES = 320000

NC = 2
NS = 16
NW = NC * NS
E_PER_W = N_EDGES // NW
CHUNK = 80
N_CHUNKS = E_PER_W // CHUNK
GROUPS = CHUNK // 16


_mesh = plsc.VectorSubcoreMesh(core_axis_name="c", subcore_axis_name="s")


@functools.partial(
    pl.kernel,
    mesh=_mesh,
    out_type=jax.ShapeDtypeStruct((N_EDGES,), jnp.float32),
    scratch_types=[
        pltpu.VMEM((CHUNK,), jnp.int32),
        pltpu.VMEM((CHUNK,), jnp.int32),
        pltpu.VMEM((CHUNK, D_FEAT), jnp.float32),
        pltpu.VMEM((CHUNK, D_FEAT), jnp.float32),
        pltpu.VMEM((CHUNK,), jnp.float32),
        pltpu.SemaphoreType.DMA,
        pltpu.SemaphoreType.DMA,
    ],
)
def _gae_decode(z_hbm, src_hbm, dst_hbm, out_hbm,
                sidx, didx, srows, drows, obuf, sem_s, sem_d):
    wid = lax.axis_index("s") * NC + lax.axis_index("c")
    base_w = wid * E_PER_W

    def chunk_body(ci, carry):
        base = base_w + ci * CHUNK
        pltpu.sync_copy(src_hbm.at[pl.ds(base, CHUNK)], sidx)
        pltpu.sync_copy(dst_hbm.at[pl.ds(base, CHUNK)], didx)
        cp_s = pltpu.async_copy(z_hbm.at[sidx], srows, sem_s)
        cp_d = pltpu.async_copy(z_hbm.at[didx], drows, sem_d)
        cp_s.wait()
        cp_d.wait()

        for g in range(GROUPS):
            row16 = lax.iota(jnp.int32, 16) + g * 16

            def d_body(j, acc):
                d0 = j * 8
                a = acc
                for k in range(8):
                    col = jnp.full((16,), d0 + k, dtype=jnp.int32)
                    vs = plsc.load_gather(srows, [row16, col])
                    vd = plsc.load_gather(drows, [row16, col])
                    a = a + vs * vd
                return a

            acc = lax.fori_loop(0, D_FEAT // 8, d_body,
                                jnp.zeros((16,), jnp.float32))
            obuf[pl.ds(g * 16, 16)] = acc

        pltpu.sync_copy(obuf, out_hbm.at[pl.ds(base, CHUNK)])
        return carry

    lax.fori_loop(0, N_CHUNKS, chunk_body, 0)


def kernel(z, edge_index):
    src = edge_index[0].astype(jnp.int32)
    dst = edge_index[1].astype(jnp.int32)
    return _gae_decode(z, src, dst)


# speedup vs baseline: 1.1019x; 1.1019x over previous
import functools

import jax
import jax.numpy as jnp
from jax import lax
from jax.experimental import pallas as pl
from jax.experimental.pallas import tpu as pltpu
from jax.experimental.pallas import tpu_sc as plsc

N_NODES = 10000
D_FEAT = 128
N_EDGES = 320000

NC = 2
NS = 16
NW = NC * NS
E_PER_W = N_EDGES // NW
CHUNK = 80
N_CHUNKS = E_PER_W // CHUNK
GROUPS = CHUNK // 16


_mesh = plsc.VectorSubcoreMesh(core_axis_name="c", subcore_axis_name="s")


@functools.partial(
    pl.kernel,
    mesh=_mesh,
    out_type=jax.ShapeDtypeStruct((N_EDGES,), jnp.float32),
    scratch_types=[
        pltpu.VMEM((CHUNK,), jnp.int32),
        pltpu.VMEM((CHUNK,), jnp.int32),
        pltpu.VMEM((CHUNK, D_FEAT), jnp.float32),
        pltpu.VMEM((CHUNK, D_FEAT), jnp.float32),
        pltpu.VMEM((CHUNK,), jnp.float32),
        pltpu.SemaphoreType.DMA,
        pltpu.SemaphoreType.DMA,
    ],
    compiler_params=pltpu.CompilerParams(needs_layout_passes=False),
)
def _gae_decode(z_hbm, src_hbm, dst_hbm, out_hbm,
                sidx, didx, srows, drows, obuf, sem_s, sem_d):
    wid = lax.axis_index("s") * NC + lax.axis_index("c")
    base_w = wid * E_PER_W

    def chunk_body(ci, carry):
        base = base_w + ci * CHUNK
        pltpu.sync_copy(src_hbm.at[pl.ds(base, CHUNK)], sidx)
        pltpu.sync_copy(dst_hbm.at[pl.ds(base, CHUNK)], didx)
        cp_s = pltpu.async_copy(z_hbm.at[sidx], srows, sem_s)
        cp_d = pltpu.async_copy(z_hbm.at[didx], drows, sem_d)
        cp_s.wait()
        cp_d.wait()

        for g in range(GROUPS):
            row16 = lax.iota(jnp.int32, 16) + g * 16

            def d_body(j, acc):
                d0 = j * 8
                a = acc
                for k in range(8):
                    col = jnp.full((16,), d0 + k, dtype=jnp.int32)
                    vs = plsc.load_gather(srows, [row16, col])
                    vd = plsc.load_gather(drows, [row16, col])
                    a = a + vs * vd
                return a

            acc = lax.fori_loop(0, D_FEAT // 8, d_body,
                                jnp.zeros((16,), jnp.float32))
            obuf[pl.ds(g * 16, 16)] = acc

        pltpu.sync_copy(obuf, out_hbm.at[pl.ds(base, CHUNK)])
        return carry

    lax.fori_loop(0, N_CHUNKS, chunk_body, 0)


def kernel(z, edge_index):
    src = edge_index[0].astype(jnp.int32)
    dst = edge_index[1].astype(jnp.int32)
    return _gae_decode(z, src, dst)


# speedup vs baseline: 1.3432x; 1.2190x over previous
import functools

import jax
import jax.numpy as jnp
from jax import lax
from jax.experimental import pallas as pl
from jax.experimental.pallas import tpu as pltpu
from jax.experimental.pallas import tpu_sc as plsc

N_NODES = 10000
D_FEAT = 128
N_EDGES = 320000

NC = 2
NS = 16
NW = NC * NS
E_PER_W = N_EDGES // NW
CHUNK = 80
N_CHUNKS = E_PER_W // CHUNK
GROUPS = CHUNK // 16


_mesh = plsc.VectorSubcoreMesh(core_axis_name="c", subcore_axis_name="s")


@functools.partial(
    pl.kernel,
    mesh=_mesh,
    out_type=jax.ShapeDtypeStruct((N_EDGES,), jnp.float32),
    scratch_types=[
        pltpu.VMEM((E_PER_W,), jnp.int32),
        pltpu.VMEM((E_PER_W,), jnp.int32),
        pltpu.VMEM((CHUNK, D_FEAT), jnp.float32),
        pltpu.VMEM((CHUNK, D_FEAT), jnp.float32),
        pltpu.VMEM((CHUNK, D_FEAT), jnp.float32),
        pltpu.VMEM((CHUNK, D_FEAT), jnp.float32),
        pltpu.VMEM((E_PER_W,), jnp.float32),
        pltpu.SemaphoreType.DMA,
        pltpu.SemaphoreType.DMA,
        pltpu.SemaphoreType.DMA,
        pltpu.SemaphoreType.DMA,
    ],
    compiler_params=pltpu.CompilerParams(needs_layout_passes=False),
)
def _gae_decode(z_hbm, src_hbm, dst_hbm, out_hbm,
                sidx, didx, s0, s1, d0, d1, obuf,
                sem_s0, sem_s1, sem_d0, sem_d1):
    wid = lax.axis_index("s") * NC + lax.axis_index("c")
    base_w = wid * E_PER_W

    pltpu.sync_copy(src_hbm.at[pl.ds(base_w, E_PER_W)], sidx)
    pltpu.sync_copy(dst_hbm.at[pl.ds(base_w, E_PER_W)], didx)

    sbufs = (s0, s1)
    dbufs = (d0, d1)
    ssems = (sem_s0, sem_s1)
    dsems = (sem_d0, sem_d1)

    def start(c, b):
        pltpu.async_copy(z_hbm.at[sidx.at[pl.ds(c * CHUNK, CHUNK)]],
                         sbufs[b], ssems[b])
        pltpu.async_copy(z_hbm.at[didx.at[pl.ds(c * CHUNK, CHUNK)]],
                         dbufs[b], dsems[b])

    def drain(b):
        pltpu.make_async_copy(z_hbm.at[sidx.at[pl.ds(0, CHUNK)]],
                              sbufs[b], ssems[b]).wait()
        pltpu.make_async_copy(z_hbm.at[didx.at[pl.ds(0, CHUNK)]],
                              dbufs[b], dsems[b]).wait()

    def compute(c, b):
        sb, db = sbufs[b], dbufs[b]

        def g_body(g, carry):
            row16 = lax.iota(jnp.int32, 16) + g * 16
            acc = jnp.zeros((16,), jnp.float32)
            for dd in range(D_FEAT):
                col = jnp.full((16,), dd, jnp.int32)
                vs = plsc.load_gather(sb, [row16, col])
                vd = plsc.load_gather(db, [row16, col])
                acc = acc + vs * vd
            obuf[pl.ds(c * CHUNK + g * 16, 16)] = acc
            return carry

        lax.fori_loop(0, GROUPS, g_body, 0)

    start(0, 0)
    start(1, 1)

    def pair_body(p, carry):
        for b in range(2):
            c = p * 2 + b

            def live():
                drain(b)
                compute(c, b)
                pl.when(c + 2 < N_CHUNKS)(lambda: start(c + 2, b))

            pl.when(c < N_CHUNKS)(live)
        return carry

    lax.fori_loop(0, (N_CHUNKS + 1) // 2, pair_body, 0)

    pltpu.sync_copy(obuf, out_hbm.at[pl.ds(base_w, E_PER_W)])


def kernel(z, edge_index):
    src = edge_index[0].astype(jnp.int32)
    dst = edge_index[1].astype(jnp.int32)
    return _gae_decode(z, src, dst)
